# baseline (speedup 1.0000x reference)
import functools

import jax
import jax.numpy as jnp
from jax import lax
from jax.experimental import pallas as pl
from jax.experimental.pallas import tpu as pltpu
from jax.experimental.pallas import tpu_sc as plsc

D_MODEL = 1024
NUM_EXPERT = 8
TOP_K = 2
N_TOKENS = 32768
NOISE_EPS = 0.01

NUM_HALVES = 2
N_HALF = N_TOKENS // NUM_HALVES
NUM_WORKERS = 32
CHUNK = N_HALF // NUM_WORKERS
LANES = 16
GROUPS = CHUNK // LANES
BLOCK_T = 1024

_INV_SQRT2 = 0.7071067811865476
_P = 0.47047
_A1 = 0.3480242
_A2 = -0.0958798
_A3 = 0.7478556


def _dense_body(wct_ref, inp_ref, slab_ref):
    logits = lax.dot_general(
        wct_ref[...], inp_ref[...],
        dimension_numbers=(((1,), (1,)), ((), ())),
        preferred_element_type=jnp.float32)
    clean = logits[:NUM_EXPERT, :]
    raw = logits[NUM_EXPERT:, :]
    stddev = jax.nn.softplus(raw) + NOISE_EPS
    slab_ref[...] = jnp.concatenate([clean, stddev], axis=0)


def _cdf(z):
    x = z * _INV_SQRT2
    ax = jnp.abs(x)
    t = 1.0 / (1.0 + _P * ax)
    poly = ((_A3 * t + _A2) * t + _A1) * t
    erf_abs = 1.0 - poly * jnp.exp(-ax * ax)
    erf = jnp.where(x >= 0.0, erf_abs, -erf_abs)
    return 0.5 * (1.0 + erf)


def _route_body(h, slab_hbm, noise_hbm, idx_hbm, gate_hbm, part_hbm,
                slab_v, noise_v, idx_v, gate_v, part_v):
    wid = lax.axis_index("c") * 16 + lax.axis_index("s")
    pltpu.sync_copy(slab_hbm.at[:, pl.ds(wid * CHUNK, CHUNK)], slab_v)
    pltpu.sync_copy(
        noise_hbm.at[:, pl.ds(h * N_HALF + wid * CHUNK, CHUNK)], noise_v)

    lane = lax.broadcasted_iota(jnp.int32, (LANES,), 0)
    lane2 = lane + lane
    zeros = jnp.zeros((LANES,), jnp.float32)
    neg = jnp.full((LANES,), -jnp.inf, jnp.float32)

    def body(g, accs):
        sl = pl.ds(g * LANES, LANES)
        clean = [slab_v[e, sl] for e in range(NUM_EXPERT)]
        std = [slab_v[NUM_EXPERT + e, sl] for e in range(NUM_EXPERT)]
        noisy = [clean[e] + std[e] * noise_v[e, sl] for e in range(NUM_EXPERT)]

        big = jnp.full((LANES,), NUM_EXPERT, jnp.int32)

        def argtop(vals):
            v = vals[0]
            for e in range(1, NUM_EXPERT):
                v = jnp.maximum(v, vals[e])
            ix = big
            for e in range(NUM_EXPERT - 1, -1, -1):
                ix = jnp.where(vals[e] == v, jnp.full((LANES,), e, jnp.int32), ix)
            return v, ix

        v1, i1 = argtop(noisy)
        m2 = [jnp.where(i1 == e, neg, noisy[e]) for e in range(NUM_EXPERT)]
        v2, i2 = argtop(m2)
        m3 = [jnp.where(i2 == e, neg, m2[e]) for e in range(NUM_EXPERT)]
        v3 = m3[0]
        for e in range(1, NUM_EXPERT):
            v3 = jnp.maximum(v3, m3[e])

        a = jnp.exp(v2 - v1)
        g1 = 1.0 / (1.0 + a)
        g2 = 1.0 - g1

        pos1 = lane2 + g * (2 * LANES)
        pos2 = pos1 + 1
        plsc.store_scatter(idx_v, [pos1], i1)
        plsc.store_scatter(idx_v, [pos2], i2)
        gate_v[0, sl] = g1
        gate_v[1, sl] = g2

        out = []
        for e in range(NUM_EXPERT):
            inv_std = 1.0 / std[e]
            thr = jnp.where(noisy[e] > v3, v3, v2)
            prob = _cdf((clean[e] - thr) * inv_std)
            imp_e = jnp.where(i1 == e, g1, zeros) + jnp.where(i2 == e, g2, zeros)
            out.append(accs[e] + imp_e)
            out.append(accs[NUM_EXPERT + e] + prob)
        return tuple(out[0::2] + out[1::2])

    init = tuple(zeros for _ in range(2 * NUM_EXPERT))
    accs = lax.fori_loop(0, GROUPS, body, init)

    part = zeros
    for e in range(2 * NUM_EXPERT):
        s = lax.reduce_sum_p.bind(accs[e], axes=(0,))
        part = part + jnp.where(lane == e, jnp.full((LANES,), 1.0) * s, zeros)
    part_v[...] = part

    pltpu.sync_copy(idx_v, idx_hbm.at[pl.ds(wid * TOP_K * CHUNK, TOP_K * CHUNK)])
    pltpu.sync_copy(gate_v, gate_hbm.at[:, pl.ds(wid * CHUNK, CHUNK)])
    pltpu.sync_copy(part_v, part_hbm.at[wid])


def _loss_body(p0_ref, p1_ref, loss_ref):
    tot = (jnp.sum(p0_ref[...], axis=0, keepdims=True)
           + jnp.sum(p1_ref[...], axis=0, keepdims=True))
    imp = tot[:, :NUM_EXPERT]
    load = tot[:, NUM_EXPERT:]

    def cv_sq(x):
        mean = jnp.mean(x, keepdims=True)
        var = jnp.sum((x - mean) ** 2, keepdims=True) / (NUM_EXPERT - 1)
        return var / (mean * mean + 1e-10)

    loss_ref[...] = cv_sq(imp) + cv_sq(load)


def _dense_half(wct, inp, h):
    grid = N_HALF // BLOCK_T
    off = h * grid
    return pl.pallas_call(
        _dense_body,
        grid=(grid,),
        in_specs=[
            pl.BlockSpec((2 * NUM_EXPERT, D_MODEL), lambda i: (0, 0)),
            pl.BlockSpec((BLOCK_T, D_MODEL), lambda i: (off + i, 0)),
        ],
        out_specs=pl.BlockSpec((2 * NUM_EXPERT, BLOCK_T), lambda i: (0, i)),
        out_shape=jax.ShapeDtypeStruct((2 * NUM_EXPERT, N_HALF), jnp.float32),
    )(wct, inp)


def _route_half(slabs, noise_t, h):
    mesh = plsc.VectorSubcoreMesh(core_axis_name="c", subcore_axis_name="s")
    route = functools.partial(
        pl.kernel,
        mesh=mesh,
        compiler_params=pltpu.CompilerParams(needs_layout_passes=False),
        out_type=[
            jax.ShapeDtypeStruct((TOP_K * N_HALF,), jnp.int32),
            jax.ShapeDtypeStruct((TOP_K, N_HALF), jnp.float32),
            jax.ShapeDtypeStruct((NUM_WORKERS, LANES), jnp.float32),
        ],
        scratch_types=[
            pltpu.VMEM((2 * NUM_EXPERT, CHUNK), jnp.float32),
            pltpu.VMEM((NUM_EXPERT, CHUNK), jnp.float32),
            pltpu.VMEM((TOP_K * CHUNK,), jnp.int32),
            pltpu.VMEM((TOP_K, CHUNK), jnp.float32),
            pltpu.VMEM((LANES,), jnp.float32),
        ],
    )(functools.partial(_route_body, h))
    return route(slabs, noise_t)


@jax.jit
def kernel(inp, w_gate, w_noise, noise):
    wct = jnp.concatenate([w_gate.T, w_noise.T], axis=0)
    noise_t = noise.T

    idxs, gates, parts = [], [], []
    for h in range(NUM_HALVES):
        slabs = _dense_half(wct, inp, h)
        idx32, gates32, part = _route_half(slabs, noise_t, h)
        idxs.append(idx32)
        gates.append(gates32)
        parts.append(part)

    loss = pl.pallas_call(
        _loss_body,
        out_shape=jax.ShapeDtypeStruct((1, 1), jnp.float32),
    )(parts[0], parts[1])

    top_k_indices = jnp.concatenate(idxs)
    gfull = jnp.concatenate(gates, axis=1)
    top_k_gates = gfull.T[:, None, :]
    return top_k_indices, top_k_gates, loss.reshape(())


# speedup vs baseline: 1.0706x; 1.0706x over previous
import functools

import jax
import jax.numpy as jnp
from jax import lax
from jax.experimental import pallas as pl
from jax.experimental.pallas import tpu as pltpu
from jax.experimental.pallas import tpu_sc as plsc

D_MODEL = 1024
NUM_EXPERT = 8
TOP_K = 2
N_TOKENS = 32768
NOISE_EPS = 0.01

SIZES = (12288, 12288, 8192)
OFFS = (0, 12288, 24576)
NUM_WORKERS = 32
LANES = 16
BLOCK_T = 2048

_INV_SQRT2 = 0.7071067811865476
_P = 0.47047
_A1 = 0.3480242
_A2 = -0.0958798
_A3 = 0.7478556


def _dense_body(wct_ref, inp_ref, slab_ref):
    logits = lax.dot_general(
        wct_ref[...], inp_ref[...],
        dimension_numbers=(((1,), (1,)), ((), ())),
        preferred_element_type=jnp.float32)
    clean = logits[:NUM_EXPERT, :]
    raw = logits[NUM_EXPERT:, :]
    stddev = jax.nn.softplus(raw) + NOISE_EPS
    slab_ref[...] = jnp.concatenate([clean, stddev], axis=0)


def _cdf(z):
    x = z * _INV_SQRT2
    ax = jnp.abs(x)
    t = 1.0 / (1.0 + _P * ax)
    poly = ((_A3 * t + _A2) * t + _A1) * t
    erf_abs = 1.0 - poly * jnp.exp(-ax * ax)
    erf = jnp.where(x >= 0.0, erf_abs, -erf_abs)
    return 0.5 * (1.0 + erf)


def _route_body(off, chunk, slab_hbm, noise_hbm, idx_hbm, gate_hbm, part_hbm,
                slab_v, noise_v, idx_v, gate_v, part_v):
    groups = chunk // LANES
    wid = lax.axis_index("c") * 16 + lax.axis_index("s")
    pltpu.sync_copy(slab_hbm.at[:, pl.ds(wid * chunk, chunk)], slab_v)
    pltpu.sync_copy(
        noise_hbm.at[:, pl.ds(off + wid * chunk, chunk)], noise_v)

    lane = lax.broadcasted_iota(jnp.int32, (LANES,), 0)
    lane2 = lane + lane
    zeros = jnp.zeros((LANES,), jnp.float32)
    neg = jnp.full((LANES,), -jnp.inf, jnp.float32)

    def body(g, accs):
        sl = pl.ds(g * LANES, LANES)
        clean = [slab_v[e, sl] for e in range(NUM_EXPERT)]
        std = [slab_v[NUM_EXPERT + e, sl] for e in range(NUM_EXPERT)]
        noisy = [clean[e] + std[e] * noise_v[e, sl] for e in range(NUM_EXPERT)]

        big = jnp.full((LANES,), NUM_EXPERT, jnp.int32)

        def argtop(vals):
            v = vals[0]
            for e in range(1, NUM_EXPERT):
                v = jnp.maximum(v, vals[e])
            ix = big
            for e in range(NUM_EXPERT - 1, -1, -1):
                ix = jnp.where(vals[e] == v, jnp.full((LANES,), e, jnp.int32), ix)
            return v, ix

        v1, i1 = argtop(noisy)
        m2 = [jnp.where(i1 == e, neg, noisy[e]) for e in range(NUM_EXPERT)]
        v2, i2 = argtop(m2)
        m3 = [jnp.where(i2 == e, neg, m2[e]) for e in range(NUM_EXPERT)]
        v3 = m3[0]
        for e in range(1, NUM_EXPERT):
            v3 = jnp.maximum(v3, m3[e])

        a = jnp.exp(v2 - v1)
        g1 = 1.0 / (1.0 + a)
        g2 = 1.0 - g1

        pos1 = lane2 + g * (2 * LANES)
        pos2 = pos1 + 1
        plsc.store_scatter(idx_v, [pos1], i1)
        plsc.store_scatter(idx_v, [pos2], i2)
        gate_v[0, sl] = g1
        gate_v[1, sl] = g2

        out = []
        for e in range(NUM_EXPERT):
            inv_std = 1.0 / std[e]
            thr = jnp.where(noisy[e] > v3, v3, v2)
            prob = _cdf((clean[e] - thr) * inv_std)
            imp_e = jnp.where(i1 == e, g1, zeros) + jnp.where(i2 == e, g2, zeros)
            out.append(accs[e] + imp_e)
            out.append(accs[NUM_EXPERT + e] + prob)
        return tuple(out[0::2] + out[1::2])

    init = tuple(zeros for _ in range(2 * NUM_EXPERT))
    accs = lax.fori_loop(0, groups, body, init)

    part = zeros
    for e in range(2 * NUM_EXPERT):
        s = lax.reduce_sum_p.bind(accs[e], axes=(0,))
        part = part + jnp.where(lane == e, jnp.full((LANES,), 1.0) * s, zeros)
    part_v[...] = part

    pltpu.sync_copy(idx_v, idx_hbm.at[pl.ds(wid * TOP_K * chunk, TOP_K * chunk)])
    pltpu.sync_copy(gate_v, gate_hbm.at[:, pl.ds(wid * chunk, chunk)])
    pltpu.sync_copy(part_v, part_hbm.at[wid])


def _loss_body(p0_ref, p1_ref, p2_ref, loss_ref):
    tot = (jnp.sum(p0_ref[...], axis=0, keepdims=True)
           + jnp.sum(p1_ref[...], axis=0, keepdims=True)
           + jnp.sum(p2_ref[...], axis=0, keepdims=True))
    imp = tot[:, :NUM_EXPERT]
    load = tot[:, NUM_EXPERT:]

    def cv_sq(x):
        mean = jnp.mean(x, keepdims=True)
        var = jnp.sum((x - mean) ** 2, keepdims=True) / (NUM_EXPERT - 1)
        return var / (mean * mean + 1e-10)

    loss_ref[...] = cv_sq(imp) + cv_sq(load)


def _dense_chunk(wct, inp, off_tok, n_tok):
    grid = n_tok // BLOCK_T
    off = off_tok // BLOCK_T
    return pl.pallas_call(
        _dense_body,
        grid=(grid,),
        in_specs=[
            pl.BlockSpec((2 * NUM_EXPERT, D_MODEL), lambda i: (0, 0)),
            pl.BlockSpec((BLOCK_T, D_MODEL), lambda i: (off + i, 0)),
        ],
        out_specs=pl.BlockSpec((2 * NUM_EXPERT, BLOCK_T), lambda i: (0, i)),
        out_shape=jax.ShapeDtypeStruct((2 * NUM_EXPERT, n_tok), jnp.float32),
    )(wct, inp)


def _route_chunk(slabs, noise_t, off_tok, n_tok):
    chunk = n_tok // NUM_WORKERS
    mesh = plsc.VectorSubcoreMesh(core_axis_name="c", subcore_axis_name="s")
    route = functools.partial(
        pl.kernel,
        mesh=mesh,
        compiler_params=pltpu.CompilerParams(needs_layout_passes=False),
        out_type=[
            jax.ShapeDtypeStruct((TOP_K * n_tok,), jnp.int32),
            jax.ShapeDtypeStruct((TOP_K, n_tok), jnp.float32),
            jax.ShapeDtypeStruct((NUM_WORKERS, LANES), jnp.float32),
        ],
        scratch_types=[
            pltpu.VMEM((2 * NUM_EXPERT, chunk), jnp.float32),
            pltpu.VMEM((NUM_EXPERT, chunk), jnp.float32),
            pltpu.VMEM((TOP_K * chunk,), jnp.int32),
            pltpu.VMEM((TOP_K, chunk), jnp.float32),
            pltpu.VMEM((LANES,), jnp.float32),
        ],
    )(functools.partial(_route_body, off_tok, chunk))
    return route(slabs, noise_t)


@jax.jit
def kernel(inp, w_gate, w_noise, noise):
    wct = jnp.concatenate([w_gate.T, w_noise.T], axis=0)
    noise_t = noise.T

    idxs, gates, parts = [], [], []
    for off, n in zip(OFFS, SIZES):
        slabs = _dense_chunk(wct, inp, off, n)
        idx32, gates32, part = _route_chunk(slabs, noise_t, off, n)
        idxs.append(idx32)
        gates.append(gates32)
        parts.append(part)

    loss = pl.pallas_call(
        _loss_body,
        out_shape=jax.ShapeDtypeStruct((1, 1), jnp.float32),
    )(*parts)

    top_k_indices = jnp.concatenate(idxs)
    gfull = jnp.concatenate(gates, axis=1)
    top_k_gates = gfull.T[:, None, :]
    return top_k_indices, top_k_gates, loss.reshape(())


# speedup vs baseline: 1.0947x; 1.0225x over previous
import functools

import jax
import jax.numpy as jnp
from jax import lax
from jax.experimental import pallas as pl
from jax.experimental.pallas import tpu as pltpu
from jax.experimental.pallas import tpu_sc as plsc

D_MODEL = 1024
NUM_EXPERT = 8
TOP_K = 2
N_TOKENS = 32768
NOISE_EPS = 0.01

SIZES = (20480, 12288)
OFFS = (0, 20480)
NUM_WORKERS = 32
LANES = 16
BLOCK_T = 2048

_INV_SQRT2 = 0.7071067811865476
_P = 0.47047
_A1 = 0.3480242
_A2 = -0.0958798
_A3 = 0.7478556


def _dense_body(wct_ref, inp_ref, slab_ref):
    logits = lax.dot_general(
        wct_ref[...], inp_ref[...],
        dimension_numbers=(((1,), (1,)), ((), ())),
        preferred_element_type=jnp.float32)
    clean = logits[:NUM_EXPERT, :]
    raw = logits[NUM_EXPERT:, :]
    stddev = jax.nn.softplus(raw) + NOISE_EPS
    slab_ref[...] = jnp.concatenate([clean, stddev], axis=0)


def _cdf(z):
    x = z * _INV_SQRT2
    ax = jnp.abs(x)
    t = 1.0 / (1.0 + _P * ax)
    poly = ((_A3 * t + _A2) * t + _A1) * t
    erf_abs = 1.0 - poly * jnp.exp(-ax * ax)
    erf = jnp.where(x >= 0.0, erf_abs, -erf_abs)
    return 0.5 * (1.0 + erf)


def _route_body(off, chunk, slab_hbm, noise_hbm, idx_hbm, gate_hbm, part_hbm,
                slab_v, noise_v, idx_v, gate_v, part_v):
    groups = chunk // LANES
    wid = lax.axis_index("c") * 16 + lax.axis_index("s")
    pltpu.sync_copy(slab_hbm.at[:, pl.ds(wid * chunk, chunk)], slab_v)
    pltpu.sync_copy(
        noise_hbm.at[:, pl.ds(off + wid * chunk, chunk)], noise_v)

    lane = lax.broadcasted_iota(jnp.int32, (LANES,), 0)
    lane2 = lane + lane
    zeros = jnp.zeros((LANES,), jnp.float32)
    neg = jnp.full((LANES,), -jnp.inf, jnp.float32)

    def body(g, accs):
        sl = pl.ds(g * LANES, LANES)
        clean = [slab_v[e, sl] for e in range(NUM_EXPERT)]
        std = [slab_v[NUM_EXPERT + e, sl] for e in range(NUM_EXPERT)]
        noisy = [clean[e] + std[e] * noise_v[e, sl] for e in range(NUM_EXPERT)]

        big = jnp.full((LANES,), NUM_EXPERT, jnp.int32)

        def argtop(vals):
            v = vals[0]
            for e in range(1, NUM_EXPERT):
                v = jnp.maximum(v, vals[e])
            ix = big
            for e in range(NUM_EXPERT - 1, -1, -1):
                ix = jnp.where(vals[e] == v, jnp.full((LANES,), e, jnp.int32), ix)
            return v, ix

        v1, i1 = argtop(noisy)
        m2 = [jnp.where(i1 == e, neg, noisy[e]) for e in range(NUM_EXPERT)]
        v2, i2 = argtop(m2)
        m3 = [jnp.where(i2 == e, neg, m2[e]) for e in range(NUM_EXPERT)]
        v3 = m3[0]
        for e in range(1, NUM_EXPERT):
            v3 = jnp.maximum(v3, m3[e])

        a = jnp.exp(v2 - v1)
        g1 = 1.0 / (1.0 + a)
        g2 = 1.0 - g1

        pos1 = lane2 + g * (2 * LANES)
        pos2 = pos1 + 1
        plsc.store_scatter(idx_v, [pos1], i1)
        plsc.store_scatter(idx_v, [pos2], i2)
        gate_v[0, sl] = g1
        gate_v[1, sl] = g2

        out = []
        for e in range(NUM_EXPERT):
            inv_std = 1.0 / std[e]
            thr = jnp.where(noisy[e] > v3, v3, v2)
            prob = _cdf((clean[e] - thr) * inv_std)
            imp_e = jnp.where(i1 == e, g1, zeros) + jnp.where(i2 == e, g2, zeros)
            out.append(accs[e] + imp_e)
            out.append(accs[NUM_EXPERT + e] + prob)
        return tuple(out[0::2] + out[1::2])

    init = tuple(zeros for _ in range(2 * NUM_EXPERT))
    accs = lax.fori_loop(0, groups, body, init)

    part = zeros
    for e in range(2 * NUM_EXPERT):
        s = lax.reduce_sum_p.bind(accs[e], axes=(0,))
        part = part + jnp.where(lane == e, jnp.full((LANES,), 1.0) * s, zeros)
    part_v[...] = part

    pltpu.sync_copy(idx_v, idx_hbm.at[pl.ds(wid * TOP_K * chunk, TOP_K * chunk)])
    pltpu.sync_copy(gate_v, gate_hbm.at[:, pl.ds(wid * chunk, chunk)])
    pltpu.sync_copy(part_v, part_hbm.at[wid])


def _loss_body(*refs):
    *part_refs, loss_ref = refs
    tot = sum(jnp.sum(p[...], axis=0, keepdims=True) for p in part_refs)
    imp = tot[:, :NUM_EXPERT]
    load = tot[:, NUM_EXPERT:]

    def cv_sq(x):
        mean = jnp.mean(x, keepdims=True)
        var = jnp.sum((x - mean) ** 2, keepdims=True) / (NUM_EXPERT - 1)
        return var / (mean * mean + 1e-10)

    loss_ref[...] = cv_sq(imp) + cv_sq(load)


def _dense_chunk(wct, inp, off_tok, n_tok):
    grid = n_tok // BLOCK_T
    off = off_tok // BLOCK_T
    return pl.pallas_call(
        _dense_body,
        grid=(grid,),
        in_specs=[
            pl.BlockSpec((2 * NUM_EXPERT, D_MODEL), lambda i: (0, 0)),
            pl.BlockSpec((BLOCK_T, D_MODEL), lambda i: (off + i, 0)),
        ],
        out_specs=pl.BlockSpec((2 * NUM_EXPERT, BLOCK_T), lambda i: (0, i)),
        out_shape=jax.ShapeDtypeStruct((2 * NUM_EXPERT, n_tok), jnp.float32),
    )(wct, inp)


def _route_chunk(slabs, noise_t, off_tok, n_tok):
    chunk = n_tok // NUM_WORKERS
    mesh = plsc.VectorSubcoreMesh(core_axis_name="c", subcore_axis_name="s")
    route = functools.partial(
        pl.kernel,
        mesh=mesh,
        compiler_params=pltpu.CompilerParams(needs_layout_passes=False),
        out_type=[
            jax.ShapeDtypeStruct((TOP_K * n_tok,), jnp.int32),
            jax.ShapeDtypeStruct((TOP_K, n_tok), jnp.float32),
            jax.ShapeDtypeStruct((NUM_WORKERS, LANES), jnp.float32),
        ],
        scratch_types=[
            pltpu.VMEM((2 * NUM_EXPERT, chunk), jnp.float32),
            pltpu.VMEM((NUM_EXPERT, chunk), jnp.float32),
            pltpu.VMEM((TOP_K * chunk,), jnp.int32),
            pltpu.VMEM((TOP_K, chunk), jnp.float32),
            pltpu.VMEM((LANES,), jnp.float32),
        ],
    )(functools.partial(_route_body, off_tok, chunk))
    return route(slabs, noise_t)


@jax.jit
def kernel(inp, w_gate, w_noise, noise):
    wct = jnp.concatenate([w_gate.T, w_noise.T], axis=0)
    noise_t = noise.T

    idxs, gates, parts = [], [], []
    for off, n in zip(OFFS, SIZES):
        slabs = _dense_chunk(wct, inp, off, n)
        idx32, gates32, part = _route_chunk(slabs, noise_t, off, n)
        idxs.append(idx32)
        gates.append(gates32)
        parts.append(part)

    loss = pl.pallas_call(
        _loss_body,
        out_shape=jax.ShapeDtypeStruct((1, 1), jnp.float32),
    )(*parts)

    top_k_indices = jnp.concatenate(idxs)
    gfull = jnp.concatenate(gates, axis=1)
    top_k_gates = gfull.T[:, None, :]
    return top_k_indices, top_k_gates, loss.reshape(())
